# baseline (speedup 1.0000x reference)
import functools

import jax
import jax.numpy as jnp
from jax import lax
from jax.experimental import pallas as pl
from jax.experimental.pallas import tpu as pltpu
from jax.experimental.pallas import tpu_sc as plsc

BATCH = 16384
DIM = 64
NC = 2
NS = 16
NW = NC * NS
BPW = BATCH // NW
LANES = 16
NGROUPS = BPW // LANES

_mesh = plsc.VectorSubcoreMesh(core_axis_name="c", subcore_axis_name="s")


def _sqrt16(x):
    i = plsc.bitcast(x, jnp.int32)
    i = jnp.int32(0x5F3759DF) - lax.shift_right_arithmetic(i, 1)
    y = plsc.bitcast(i, jnp.float32)
    for _ in range(3):
        y = y * (jnp.float32(1.5) - jnp.float32(0.5) * x * y * y)
    return x * y


@functools.partial(
    pl.kernel,
    out_type=jax.ShapeDtypeStruct((BATCH,), jnp.float32),
    mesh=_mesh,
    compiler_params=pltpu.CompilerParams(
        needs_layout_passes=False, use_tc_tiling_on_sc=False),
    scratch_types=[
        pltpu.VMEM((BPW,), jnp.int32),
        pltpu.VMEM((BPW,), jnp.int32),
        pltpu.VMEM((BPW,), jnp.int32),
        pltpu.VMEM((BPW,), jnp.int32),
        pltpu.VMEM((BPW, DIM), jnp.float32),
        pltpu.VMEM((BPW, DIM), jnp.float32),
        pltpu.VMEM((DIM,), jnp.float32),
        pltpu.VMEM((BPW,), jnp.float32),
        pltpu.SemaphoreType.DMA,
        pltpu.SemaphoreType.DMA,
    ],
)
def _score_kernel(xs_hbm, ys_hbm, gid_hbm, emb_hbm, rel_hbm, out_hbm,
                  xv, yv, xe, ye, hv, tv, rv, ov, sem1, sem2):
    wid = lax.axis_index("s") * NC + lax.axis_index("c")
    base = wid * BPW

    pltpu.sync_copy(xs_hbm.at[pl.ds(base, BPW)], xv)
    pltpu.sync_copy(ys_hbm.at[pl.ds(base, BPW)], yv)
    pltpu.sync_copy(rel_hbm, rv)

    cx = pltpu.async_copy(gid_hbm.at[xv], xe, sem1)
    cy = pltpu.async_copy(gid_hbm.at[yv], ye, sem2)
    cx.wait()
    cy.wait()

    ch = pltpu.async_copy(emb_hbm.at[xe], hv, sem1)
    ct = pltpu.async_copy(emb_hbm.at[ye], tv, sem2)
    ch.wait()
    ct.wait()

    rchunks = [rv[pl.ds(c * LANES, LANES)] for c in range(DIM // LANES)]
    lane_iota = lax.iota(jnp.int32, LANES)
    perms = {d: lane_iota ^ d for d in (1, 2, 4, 8)}
    masks = {d: (lane_iota & d) == 0 for d in (1, 2, 4, 8)}

    def combine(a, b, dist):
        m = masks[dist]
        w = jnp.where(m, b, a)
        _, wp = plsc.sort_key_val(perms[dist], w)
        return jnp.where(m, a, b) + wp

    def group_body(g, carry):
        svecs = []
        for p in range(LANES):
            i = g * LANES + p
            s = None
            for j in range(DIM // LANES):
                hj = hv[i, pl.ds(j * LANES, LANES)]
                tj = tv[i, pl.ds(j * LANES, LANES)]
                d = hj - tj + rchunks[j]
                s = d * d if s is None else s + d * d
            svecs.append(s)
        dist = 1
        while len(svecs) > 1:
            svecs = [combine(svecs[k], svecs[k + 1], dist)
                     for k in range(0, len(svecs), 2)]
            dist *= 2
        ov[pl.ds(g * LANES, LANES)] = _sqrt16(svecs[0])
        return carry

    lax.fori_loop(0, NGROUPS, group_body, 0)

    pltpu.sync_copy(ov, out_hbm.at[pl.ds(base, BPW)])


def kernel(data, graph_ids, entity_emb, relation_emb):
    xs = data[:, 0]
    ys = data[:, 1]
    rel = relation_emb.reshape(DIM)
    scores = _score_kernel(xs, ys, graph_ids, entity_emb, rel)
    return scores.reshape(BATCH, 1)
